# baseline (speedup 1.0000x reference)
import functools

import jax
import jax.numpy as jnp
from jax import lax
from jax.experimental import pallas as pl
from jax.experimental.pallas import tpu as pltpu
from jax.experimental.pallas import tpu_sc as plsc

_NUM_CORES = 2
_NUM_SUBCORES = 16
_NUM_WORKERS = _NUM_CORES * _NUM_SUBCORES
_LANES = 16

_DIM = 64
_SEG = 26
_BATCH = 16384
_CHUNK = 256


def _make_gather():
    b_per_w = _BATCH // _NUM_WORKERS
    n_chunks = b_per_w // _CHUNK
    total = _SEG * n_chunks

    mesh = plsc.VectorSubcoreMesh(
        core_axis_name="c",
        subcore_axis_name="s",
        num_cores=_NUM_CORES,
        num_subcores=_NUM_SUBCORES,
    )

    @functools.partial(
        pl.kernel,
        out_type=(
            jax.ShapeDtypeStruct((_SEG * _DIM, _BATCH), jnp.float32),
            jax.ShapeDtypeStruct((_DIM, _CHUNK), jnp.float32),
        ),
        mesh=mesh,
        scratch_types=[
            pltpu.VMEM((_SEG * b_per_w,), jnp.int32),
            [pltpu.VMEM((_CHUNK,), jnp.int32) for _ in range(2)],
            [pltpu.VMEM((_CHUNK, 128), jnp.float32) for _ in range(2)],
            [pltpu.VMEM((_DIM, _CHUNK), jnp.float32) for _ in range(2)],
            [pltpu.SemaphoreType.DMA for _ in range(2)],
            [pltpu.SemaphoreType.DMA for _ in range(2)],
        ],
        compiler_params=pltpu.CompilerParams(
            use_tc_tiling_on_sc=True, needs_layout_passes=False
        ),
    )
    def gather_kernel(
        wpair_hbm, idxt_hbm, out_hbm, dump_hbm,
        all_v, pidx_vs, g_vs, o_vs, gsems, osems,
    ):
        wid = lax.axis_index("s") * _NUM_CORES + lax.axis_index("c")
        b0 = wid * b_per_w
        lane = lax.iota(jnp.int32, _LANES)
        offs = [(lane + k) & (_LANES - 1) for k in range(_LANES)]

        def prep(k, pb):
            for i in range(_CHUNK // _LANES):
                sl = pl.ds(k * _CHUNK + i * _LANES, _LANES)
                pidx_vs[pb][pl.ds(i * _LANES, _LANES)] = all_v[sl] >> 1
            pltpu.async_copy(wpair_hbm.at[pidx_vs[pb]], g_vs[pb], gsems[pb])

        def wait_gather(pb):
            pltpu.make_async_copy(
                wpair_hbm.at[pidx_vs[pb]], g_vs[pb], gsems[pb]
            ).wait()

        def drain_store(pb):
            pltpu.make_async_copy(o_vs[pb], dump_hbm, osems[pb]).wait()

        def phase(k, pb):
            nk = lax.min(k + 1, total - 1)
            prep(nk, 1 - pb)
            drain_store(pb)
            wait_gather(pb)
            g_v = g_vs[pb]
            o_v = o_vs[pb]

            @plsc.parallel_loop(0, _CHUNK // _LANES, unroll=2)
            def bg_body(bg):
                half = (all_v[pl.ds(k * _CHUNK + bg * _LANES, _LANES)] & 1) * _DIM
                rows = bg * _LANES + lane
                halfoffs = [half + offs[j] for j in range(_LANES)]
                colout = bg * _LANES + lane
                for dg in range(_DIM // _LANES):
                    vecs = [
                        plsc.load_gather(g_v, [rows, halfoffs[j] + dg * _LANES])
                        for j in range(_LANES)
                    ]
                    for j in range(_LANES):
                        plsc.store_scatter(
                            o_v, [offs[j] + dg * _LANES, colout], vecs[j]
                        )
            s = k // n_chunks
            base = b0 + (k % n_chunks) * _CHUNK
            pltpu.async_copy(
                o_v,
                out_hbm.at[pl.ds(s * _DIM, _DIM), pl.ds(base, _CHUNK)],
                osems[pb],
            )

        for s in range(_SEG):
            pltpu.sync_copy(
                idxt_hbm.at[s, pl.ds(b0, b_per_w)],
                all_v.at[pl.ds(s * b_per_w, b_per_w)],
            )
        for pb in range(2):
            pltpu.async_copy(o_vs[pb], dump_hbm, osems[pb])
        prep(0, 0)

        def body(k2, carry):
            phase(2 * k2, 0)
            phase(2 * k2 + 1, 1)
            return carry

        lax.fori_loop(0, total // 2, body, 0)
        drain_store(0)
        drain_store(1)
        wait_gather(0)

    return gather_kernel


def kernel(indices, weight):
    wpair = weight.reshape(500000, 128)
    idxt = indices.T.astype(jnp.int32)
    x, _ = _make_gather()(wpair, idxt)
    return x.reshape(_SEG, _DIM, _BATCH).transpose(2, 0, 1)


# speedup vs baseline: 1.1952x; 1.1952x over previous
import functools

import jax
import jax.numpy as jnp
from jax import lax
from jax.experimental import pallas as pl
from jax.experimental.pallas import tpu as pltpu
from jax.experimental.pallas import tpu_sc as plsc

_NUM_CORES = 2
_NUM_SUBCORES = 16
_NUM_WORKERS = _NUM_CORES * _NUM_SUBCORES
_LANES = 16

_DIM = 64
_SEG = 26
_BATCH = 16384
_CHUNK = 256


def _make_gather():
    b_per_w = _BATCH // _NUM_WORKERS
    n_chunks = b_per_w // _CHUNK
    total = _SEG * n_chunks

    mesh = plsc.VectorSubcoreMesh(
        core_axis_name="c",
        subcore_axis_name="s",
        num_cores=_NUM_CORES,
        num_subcores=_NUM_SUBCORES,
    )

    @functools.partial(
        pl.kernel,
        out_type=(
            jax.ShapeDtypeStruct((_SEG * _DIM, _BATCH), jnp.float32),
            jax.ShapeDtypeStruct((_DIM, _CHUNK), jnp.float32),
        ),
        mesh=mesh,
        scratch_types=[
            pltpu.VMEM((_SEG * b_per_w,), jnp.int32),
            [pltpu.VMEM((_CHUNK,), jnp.int32) for _ in range(2)],
            [pltpu.VMEM((_CHUNK, 128), jnp.float32) for _ in range(2)],
            [pltpu.VMEM((_DIM, _CHUNK), jnp.float32) for _ in range(2)],
            [pltpu.SemaphoreType.DMA for _ in range(2)],
            [pltpu.SemaphoreType.DMA for _ in range(2)],
        ],
        compiler_params=pltpu.CompilerParams(
            use_tc_tiling_on_sc=True, needs_layout_passes=False
        ),
    )
    def gather_kernel(
        wpair_hbm, idxt_hbm, out_hbm, dump_hbm,
        all_v, pidx_vs, g_vs, o_vs, gsems, osems,
    ):
        wid = lax.axis_index("s") * _NUM_CORES + lax.axis_index("c")
        b0 = wid * b_per_w
        lane = lax.iota(jnp.int32, _LANES)
        offs = [(lane + k) & (_LANES - 1) for k in range(_LANES)]

        def prep(k, pb):
            for i in range(_CHUNK // _LANES):
                sl = pl.ds(k * _CHUNK + i * _LANES, _LANES)
                pidx_vs[pb][pl.ds(i * _LANES, _LANES)] = all_v[sl] >> 1
            pltpu.async_copy(wpair_hbm.at[pidx_vs[pb]], g_vs[pb], gsems[pb])

        def wait_gather(pb):
            pltpu.make_async_copy(
                wpair_hbm.at[pidx_vs[pb]], g_vs[pb], gsems[pb]
            ).wait()

        def drain_store(pb):
            pltpu.make_async_copy(o_vs[pb], dump_hbm, osems[pb]).wait()

        def phase(k, pb):
            nk = lax.min(k + 1, total - 1)
            prep(nk, 1 - pb)
            drain_store(pb)
            wait_gather(pb)
            g_v = g_vs[pb]
            o_v = o_vs[pb]

            def bg_body(bg, bcarry):
                half = (all_v[pl.ds(k * _CHUNK + bg * _LANES, _LANES)] & 1) * _DIM
                rows = bg * _LANES + lane
                halfoffs = [half + offs[j] for j in range(_LANES)]
                colout = bg * _LANES + lane
                for dg in range(_DIM // _LANES):
                    vecs = [
                        plsc.load_gather(g_v, [rows, halfoffs[j] + dg * _LANES])
                        for j in range(_LANES)
                    ]
                    for j in range(_LANES):
                        plsc.store_scatter(
                            o_v, [offs[j] + dg * _LANES, colout], vecs[j]
                        )
                return bcarry

            lax.fori_loop(0, _CHUNK // _LANES, bg_body, 0)
            s = k // n_chunks
            base = b0 + (k % n_chunks) * _CHUNK
            pltpu.async_copy(
                o_v,
                out_hbm.at[pl.ds(s * _DIM, _DIM), pl.ds(base, _CHUNK)],
                osems[pb],
            )

        for s in range(_SEG):
            pltpu.sync_copy(
                idxt_hbm.at[s, pl.ds(b0, b_per_w)],
                all_v.at[pl.ds(s * b_per_w, b_per_w)],
            )
        for pb in range(2):
            pltpu.async_copy(o_vs[pb], dump_hbm, osems[pb])
        prep(0, 0)

        def body(k2, carry):
            phase(2 * k2, 0)
            phase(2 * k2 + 1, 1)
            return carry

        lax.fori_loop(0, total // 2, body, 0)
        drain_store(0)
        drain_store(1)
        wait_gather(0)

    return gather_kernel


def kernel(indices, weight):
    wpair = weight.reshape(500000, 128)
    idxt = indices.T.astype(jnp.int32)
    x, _ = _make_gather()(wpair, idxt)
    return x.reshape(_SEG, _DIM, _BATCH).transpose(2, 0, 1)


# speedup vs baseline: 1.2137x; 1.0155x over previous
import functools

import jax
import jax.numpy as jnp
from jax import lax
from jax.experimental import pallas as pl
from jax.experimental.pallas import tpu as pltpu
from jax.experimental.pallas import tpu_sc as plsc

_NUM_CORES = 2
_NUM_SUBCORES = 16
_NUM_WORKERS = _NUM_CORES * _NUM_SUBCORES
_LANES = 16

_DIM = 64
_SEG = 26
_BATCH = 16384
_CHUNK = 256


def _make_gather():
    b_per_w = _BATCH // _NUM_WORKERS
    n_chunks = b_per_w // _CHUNK
    total = _SEG * n_chunks

    mesh = plsc.VectorSubcoreMesh(
        core_axis_name="c",
        subcore_axis_name="s",
        num_cores=_NUM_CORES,
        num_subcores=_NUM_SUBCORES,
    )

    @functools.partial(
        pl.kernel,
        out_type=(
            jax.ShapeDtypeStruct((_SEG * _DIM, _BATCH), jnp.float32),
            jax.ShapeDtypeStruct((_DIM, _CHUNK), jnp.float32),
        ),
        mesh=mesh,
        scratch_types=[
            pltpu.VMEM((_SEG * b_per_w,), jnp.int32),
            [pltpu.VMEM((_CHUNK,), jnp.int32) for _ in range(2)],
            [pltpu.VMEM((_CHUNK, 128), jnp.float32) for _ in range(2)],
            [pltpu.VMEM((_DIM, _CHUNK), jnp.float32) for _ in range(2)],
            [pltpu.SemaphoreType.DMA for _ in range(2)],
            [pltpu.SemaphoreType.DMA for _ in range(2)],
        ],
        compiler_params=pltpu.CompilerParams(
            use_tc_tiling_on_sc=True, needs_layout_passes=False
        ),
    )
    def gather_kernel(
        wpair_hbm, idxt_hbm, out_hbm, dump_hbm,
        all_v, pidx_vs, g_vs, o_vs, gsems, osems,
    ):
        wid = lax.axis_index("s") * _NUM_CORES + lax.axis_index("c")
        b0 = wid * b_per_w
        lane = lax.iota(jnp.int32, _LANES)
        offs = [(lane + k) & (_LANES - 1) for k in range(_LANES)]

        def prep(k, pb):
            for i in range(_CHUNK // _LANES):
                sl = pl.ds(k * _CHUNK + i * _LANES, _LANES)
                pidx_vs[pb][pl.ds(i * _LANES, _LANES)] = all_v[sl] >> 1
            pltpu.async_copy(wpair_hbm.at[pidx_vs[pb]], g_vs[pb], gsems[pb])

        def wait_gather(pb):
            pltpu.make_async_copy(
                wpair_hbm.at[pidx_vs[pb]], g_vs[pb], gsems[pb]
            ).wait()

        def drain_store(pb):
            pltpu.make_async_copy(o_vs[pb], dump_hbm, osems[pb]).wait()

        def phase(k, pb):
            nk = lax.min(k + 1, total - 1)
            prep(nk, 1 - pb)
            drain_store(pb)
            wait_gather(pb)
            g_v = g_vs[pb]
            o_v = o_vs[pb]

            def bg_body(bg2, bcarry):
                for sub in range(2):
                    bg = bg2 * 2 + sub
                    half = (
                        all_v[pl.ds(k * _CHUNK + bg * _LANES, _LANES)] & 1
                    ) * _DIM
                    rows = bg * _LANES + lane
                    halfoffs = [half + offs[j] for j in range(_LANES)]
                    colout = bg * _LANES + lane
                    for dg in range(_DIM // _LANES):
                        vecs = [
                            plsc.load_gather(
                                g_v, [rows, halfoffs[j] + dg * _LANES]
                            )
                            for j in range(_LANES)
                        ]
                        for j in range(_LANES):
                            plsc.store_scatter(
                                o_v, [offs[j] + dg * _LANES, colout], vecs[j]
                            )
                return bcarry

            lax.fori_loop(0, _CHUNK // (2 * _LANES), bg_body, 0)
            s = k // n_chunks
            base = b0 + (k % n_chunks) * _CHUNK
            pltpu.async_copy(
                o_v,
                out_hbm.at[pl.ds(s * _DIM, _DIM), pl.ds(base, _CHUNK)],
                osems[pb],
            )

        for s in range(_SEG):
            pltpu.sync_copy(
                idxt_hbm.at[s, pl.ds(b0, b_per_w)],
                all_v.at[pl.ds(s * b_per_w, b_per_w)],
            )
        for pb in range(2):
            pltpu.async_copy(o_vs[pb], dump_hbm, osems[pb])
        prep(0, 0)

        def body(k2, carry):
            phase(2 * k2, 0)
            phase(2 * k2 + 1, 1)
            return carry

        lax.fori_loop(0, total // 2, body, 0)
        drain_store(0)
        drain_store(1)
        wait_gather(0)

    return gather_kernel


def kernel(indices, weight):
    wpair = weight.reshape(500000, 128)
    idxt = indices.T.astype(jnp.int32)
    x, _ = _make_gather()(wpair, idxt)
    return x.reshape(_SEG, _DIM, _BATCH).transpose(2, 0, 1)


# speedup vs baseline: 1.5387x; 1.2678x over previous
import functools

import jax
import jax.numpy as jnp
from jax import lax
from jax.experimental import pallas as pl
from jax.experimental.pallas import tpu as pltpu
from jax.experimental.pallas import tpu_sc as plsc

_NUM_CORES = 2
_NUM_SUBCORES = 16
_NUM_WORKERS = _NUM_CORES * _NUM_SUBCORES
_LANES = 16

_DIM = 64
_SEG = 26
_BATCH = 16384
_CHUNK = 256


def _make_gather():
    b_per_w = _BATCH // _NUM_WORKERS
    n_chunks = b_per_w // _CHUNK
    total = _SEG * n_chunks

    mesh = plsc.VectorSubcoreMesh(
        core_axis_name="c",
        subcore_axis_name="s",
        num_cores=_NUM_CORES,
        num_subcores=_NUM_SUBCORES,
    )

    @functools.partial(
        pl.kernel,
        out_type=(
            jax.ShapeDtypeStruct((_SEG * _DIM, _BATCH), jnp.float32),
            jax.ShapeDtypeStruct((_DIM, _CHUNK), jnp.float32),
        ),
        mesh=mesh,
        scratch_types=[
            pltpu.VMEM((_SEG * b_per_w,), jnp.int32),
            [pltpu.VMEM((_CHUNK,), jnp.int32) for _ in range(2)],
            [pltpu.VMEM((_CHUNK, 128), jnp.float32) for _ in range(2)],
            [pltpu.VMEM((_DIM, _CHUNK), jnp.float32) for _ in range(2)],
            [pltpu.SemaphoreType.DMA for _ in range(2)],
            [pltpu.SemaphoreType.DMA for _ in range(2)],
        ],
        compiler_params=pltpu.CompilerParams(
            use_tc_tiling_on_sc=True, needs_layout_passes=False
        ),
    )
    def gather_kernel(
        wpair_hbm, idxt_hbm, out_hbm, dump_hbm,
        all_v, pidx_vs, g_vs, o_vs, gsems, osems,
    ):
        wid = lax.axis_index("s") * _NUM_CORES + lax.axis_index("c")
        b0 = wid * b_per_w
        lane = lax.iota(jnp.int32, _LANES)
        offs = [(lane + k) & (_LANES - 1) for k in range(_LANES)]

        def prep(k, pb):
            for i in range(_CHUNK // _LANES):
                sl = pl.ds(k * _CHUNK + i * _LANES, _LANES)
                pidx_vs[pb][pl.ds(i * _LANES, _LANES)] = all_v[sl] >> 1
            pltpu.async_copy(wpair_hbm.at[pidx_vs[pb]], g_vs[pb], gsems[pb])

        def wait_gather(pb):
            pltpu.make_async_copy(
                wpair_hbm.at[pidx_vs[pb]], g_vs[pb], gsems[pb]
            ).wait()

        def drain_store(pb):
            pltpu.make_async_copy(o_vs[pb], dump_hbm, osems[pb]).wait()

        def phase(k, pb):
            nk = lax.min(k + 1, total - 1)
            prep(nk, 1 - pb)
            drain_store(pb)
            wait_gather(pb)
            g_v = g_vs[pb]
            o_v = o_vs[pb]

            def bg_body(bg2, bcarry):
                for sub in range(4):
                    bg = bg2 * 4 + sub
                    half = (
                        all_v[pl.ds(k * _CHUNK + bg * _LANES, _LANES)] & 1
                    ) * _DIM
                    rows = bg * _LANES + lane
                    halfoffs = [half + offs[j] for j in range(_LANES)]
                    colout = bg * _LANES + lane
                    for dg in range(_DIM // _LANES):
                        vecs = [
                            plsc.load_gather(
                                g_v, [rows, halfoffs[j] + dg * _LANES]
                            )
                            for j in range(_LANES)
                        ]
                        for j in range(_LANES):
                            plsc.store_scatter(
                                o_v, [offs[j] + dg * _LANES, colout], vecs[j]
                            )
                return bcarry

            lax.fori_loop(0, _CHUNK // (4 * _LANES), bg_body, 0)
            s = k // n_chunks
            base = b0 + (k % n_chunks) * _CHUNK
            pltpu.async_copy(
                o_v,
                out_hbm.at[pl.ds(s * _DIM, _DIM), pl.ds(base, _CHUNK)],
                osems[pb],
            )

        for s in range(_SEG):
            pltpu.sync_copy(
                idxt_hbm.at[s, pl.ds(b0, b_per_w)],
                all_v.at[pl.ds(s * b_per_w, b_per_w)],
            )
        for pb in range(2):
            pltpu.async_copy(o_vs[pb], dump_hbm, osems[pb])
        prep(0, 0)

        def body(k2, carry):
            phase(2 * k2, 0)
            phase(2 * k2 + 1, 1)
            return carry

        lax.fori_loop(0, total // 2, body, 0)
        drain_store(0)
        drain_store(1)
        wait_gather(0)

    return gather_kernel


def kernel(indices, weight):
    wpair = weight.reshape(500000, 128)
    idxt = indices.T.astype(jnp.int32)
    x, _ = _make_gather()(wpair, idxt)
    return x.reshape(_SEG, _DIM, _BATCH).transpose(2, 0, 1)
